# baseline (speedup 1.0000x reference)
import functools

import jax
import jax.numpy as jnp
from jax import lax
from jax.experimental import pallas as pl
from jax.experimental.pallas import tpu as pltpu
from jax.experimental.pallas import tpu_sc as plsc

_NC = 2
_NS = 16
_NW = _NC * _NS
_CHUNK = 128

_MESH = dict(core_axis_name="c", subcore_axis_name="s")
_PARAMS = dict(compiler_params=pltpu.CompilerParams(use_tc_tiling_on_sc=False))


def _make_permute(n_idx, dp):
    per_w = n_idx // _NW
    n_chunks = per_w // _CHUNK

    @functools.partial(
        pl.kernel,
        mesh=plsc.VectorSubcoreMesh(**_MESH),
        out_type=jax.ShapeDtypeStruct((n_idx, dp), jnp.float32),
        scratch_types=[
            pltpu.VMEM((_CHUNK,), jnp.int32),
            pltpu.VMEM((_CHUNK, dp), jnp.float32),
            pltpu.SemaphoreType.DMA,
        ],
        **_PARAMS,
    )
    def k(src_hbm, idx_hbm, out_hbm, idx_v, rows_v, sem):
        wid = lax.axis_index("s") * _NC + lax.axis_index("c")
        base = wid * per_w

        def body(j, carry):
            off = base + j * _CHUNK
            pltpu.sync_copy(idx_hbm.at[pl.ds(off, _CHUNK)], idx_v)
            pltpu.async_copy(src_hbm.at[idx_v], rows_v, sem).wait()
            pltpu.sync_copy(rows_v, out_hbm.at[pl.ds(off, _CHUNK)])
            return carry

        lax.fori_loop(0, n_chunks, body, 0)

    return k


_SUP = 512
_IDXBLK = 4096


def _make_gather(n_tok, d, dp):
    per_w = n_tok // _NW
    n_sup = per_w // _SUP
    sup_per_blk = _IDXBLK // _SUP

    @functools.partial(
        pl.kernel,
        mesh=plsc.VectorSubcoreMesh(**_MESH),
        out_type=jax.ShapeDtypeStruct((n_tok, dp), jnp.float32),
        scratch_types=[
            pltpu.VMEM((2, _IDXBLK), jnp.int32),
            pltpu.VMEM((2, _SUP, d), jnp.float32),
            pltpu.SemaphoreType.DMA,
            pltpu.SemaphoreType.DMA,
            pltpu.SemaphoreType.DMA,
            pltpu.SemaphoreType.DMA,
        ],
        **_PARAMS,
    )
    def k(src_hbm, idx_hbm, out_hbm, idx_v, rows_v, g0, g1, o0, o1):
        wid = lax.axis_index("s") * _NC + lax.axis_index("c")
        base = wid * per_w
        sem_g = (g0, g1)
        sem_o = (o0, o1)

        def load_idx_blk(blk, ib):
            pltpu.sync_copy(
                idx_hbm.at[pl.ds(base + blk * _IDXBLK, _IDXBLK)],
                idx_v.at[ib])

        def fire_gathers(g, b):
            ib = (g // sup_per_blk) % 2
            loc = (g % sup_per_blk) * _SUP
            for j in range(_SUP // _CHUNK):
                pltpu.async_copy(
                    src_hbm.at[idx_v.at[ib, pl.ds(loc + j * _CHUNK, _CHUNK)]],
                    rows_v.at[b, pl.ds(j * _CHUNK, _CHUNK)],
                    sem_g[b],
                )

        def drain_gathers(b):
            pltpu.make_async_copy(
                src_hbm.at[pl.ds(0, _SUP)], rows_v.at[b], sem_g[b]
            ).wait()

        load_idx_blk(0, 0)
        for b in range(2):
            fire_gathers(b, b)

        def body(i, carry):
            for b in range(2):
                g = 2 * i + b
                drain_gathers(b)
                store = pltpu.async_copy(
                    rows_v.at[b],
                    out_hbm.at[pl.ds(base + g * _SUP, _SUP), pl.ds(0, d)],
                    sem_o[b])

                @pl.when(
                    jnp.logical_and(
                        (g + 2) % sup_per_blk == 0,
                        (g + 2) // sup_per_blk < n_sup // sup_per_blk))
                def _():
                    load_idx_blk((g + 2) // sup_per_blk,
                                 ((g + 2) // sup_per_blk) % 2)

                store.wait()

                @pl.when(g + 2 < n_sup)
                def _():
                    fire_gathers(g + 2, b)

            return carry

        lax.fori_loop(0, n_sup // 2, body, 0)

    return k


def kernel(input_ids, id_map, table):
    b, h = input_ids.shape
    v, d = table.shape
    ids_flat = input_ids.reshape(-1).astype(jnp.int32)
    idmap = id_map.astype(jnp.int32)

    grain = _NW * _CHUNK
    v_pad = ((v + grain - 1) // grain) * grain
    if v_pad != v:
        idmap = jnp.concatenate(
            [idmap, jnp.zeros((v_pad - v,), jnp.int32)])

    permute = _make_permute(v_pad, d)
    ptable = permute(table, idmap)

    gather = _make_gather(b * h, d, _CHUNK)
    out = gather(ptable, ids_flat)
    return out[:, :d].reshape(b, h, d)


# speedup vs baseline: 1.0025x; 1.0025x over previous
import functools

import jax
import jax.numpy as jnp
from jax import lax
from jax.experimental import pallas as pl
from jax.experimental.pallas import tpu as pltpu
from jax.experimental.pallas import tpu_sc as plsc

_NC = 2
_NS = 16
_NW = _NC * _NS
_CHUNK = 128

_MESH = dict(core_axis_name="c", subcore_axis_name="s")
_PARAMS = dict(compiler_params=pltpu.CompilerParams(use_tc_tiling_on_sc=False))


def _make_permute(n_idx, dp):
    per_w = n_idx // _NW
    n_chunks = per_w // _CHUNK

    @functools.partial(
        pl.kernel,
        mesh=plsc.VectorSubcoreMesh(**_MESH),
        out_type=jax.ShapeDtypeStruct((n_idx, dp), jnp.float32),
        scratch_types=[
            pltpu.VMEM((_CHUNK,), jnp.int32),
            pltpu.VMEM((_CHUNK, dp), jnp.float32),
            pltpu.SemaphoreType.DMA,
        ],
        **_PARAMS,
    )
    def k(src_hbm, idx_hbm, out_hbm, idx_v, rows_v, sem):
        wid = lax.axis_index("s") * _NC + lax.axis_index("c")
        base = wid * per_w

        def body(j, carry):
            off = base + j * _CHUNK
            pltpu.sync_copy(idx_hbm.at[pl.ds(off, _CHUNK)], idx_v)
            pltpu.async_copy(src_hbm.at[idx_v], rows_v, sem).wait()
            pltpu.sync_copy(rows_v, out_hbm.at[pl.ds(off, _CHUNK)])
            return carry

        lax.fori_loop(0, n_chunks, body, 0)

    return k


_SUP = 256
_IDXBLK = 2048


def _make_gather(n_tok, d, dp):
    per_w = n_tok // _NW
    n_sup = per_w // _SUP
    sup_per_blk = _IDXBLK // _SUP

    @functools.partial(
        pl.kernel,
        mesh=plsc.VectorSubcoreMesh(**_MESH),
        out_type=jax.ShapeDtypeStruct((n_tok, dp), jnp.float32),
        scratch_types=[
            pltpu.VMEM((2, _IDXBLK), jnp.int32),
            pltpu.VMEM((2, _SUP, d), jnp.float32),
            pltpu.SemaphoreType.DMA,
            pltpu.SemaphoreType.DMA,
            pltpu.SemaphoreType.DMA,
            pltpu.SemaphoreType.DMA,
        ],
        **_PARAMS,
    )
    def k(src_hbm, idx_hbm, out_hbm, idx_v, rows_v, g0, g1, o0, o1):
        wid = lax.axis_index("s") * _NC + lax.axis_index("c")
        base = wid * per_w
        sem_g = (g0, g1)
        sem_o = (o0, o1)

        def load_idx_blk(blk, ib):
            pltpu.sync_copy(
                idx_hbm.at[pl.ds(base + blk * _IDXBLK, _IDXBLK)],
                idx_v.at[ib])

        def fire_gathers(g, b):
            ib = (g // sup_per_blk) % 2
            loc = (g % sup_per_blk) * _SUP
            for j in range(_SUP // _CHUNK):
                pltpu.async_copy(
                    src_hbm.at[idx_v.at[ib, pl.ds(loc + j * _CHUNK, _CHUNK)]],
                    rows_v.at[b, pl.ds(j * _CHUNK, _CHUNK)],
                    sem_g[b],
                )

        def drain_gathers(b):
            pltpu.make_async_copy(
                src_hbm.at[pl.ds(0, _SUP)], rows_v.at[b], sem_g[b]
            ).wait()

        load_idx_blk(0, 0)
        for b in range(2):
            fire_gathers(b, b)

        def body(i, carry):
            for b in range(2):
                g = 2 * i + b
                drain_gathers(b)
                store = pltpu.async_copy(
                    rows_v.at[b],
                    out_hbm.at[pl.ds(base + g * _SUP, _SUP), pl.ds(0, d)],
                    sem_o[b])

                @pl.when(
                    jnp.logical_and(
                        (g + 2) % sup_per_blk == 0,
                        (g + 2) // sup_per_blk < n_sup // sup_per_blk))
                def _():
                    load_idx_blk((g + 2) // sup_per_blk,
                                 ((g + 2) // sup_per_blk) % 2)

                store.wait()

                @pl.when(g + 2 < n_sup)
                def _():
                    fire_gathers(g + 2, b)

            return carry

        lax.fori_loop(0, n_sup // 2, body, 0)

    return k


def kernel(input_ids, id_map, table):
    b, h = input_ids.shape
    v, d = table.shape
    ids_flat = input_ids.reshape(-1).astype(jnp.int32)
    idmap = id_map.astype(jnp.int32)

    grain = _NW * _CHUNK
    v_pad = ((v + grain - 1) // grain) * grain
    if v_pad != v:
        idmap = jnp.concatenate(
            [idmap, jnp.zeros((v_pad - v,), jnp.int32)])

    permute = _make_permute(v_pad, d)
    ptable = permute(table, idmap)

    gather = _make_gather(b * h, d, _CHUNK)
    out = gather(ptable, ids_flat)
    return out[:, :d].reshape(b, h, d)
